# baseline (speedup 1.0000x reference)
import jax
import jax.numpy as jnp
from jax import lax
from jax.experimental import pallas as pl
from jax.experimental.pallas import tpu as pltpu
from jax.experimental.pallas import tpu_sc as plsc

N = 10000
E = 160000
D = 256
HALF = 128
NC = 2
NS = 16
CW = 128
EDGES_PER_TILE = E // NS
NCH = 80
DEG_PER_WORKER = E // (NC * NS)
DNCH = 40
NROWS = N + 8
SPAN = 624
TAIL = N - SPAN * NS
BLK = 1000
NB = N // BLK


def _mm_body(x_ref, w_ref, b_ref, out_ref):
    acc = jnp.dot(x_ref[...], w_ref[...], preferred_element_type=jnp.float32)
    out_ref[...] = acc + b_ref[0]


def _support1(x, W1, b1_2h):
    return pl.pallas_call(
        _mm_body,
        grid=(NC, NB),
        in_specs=[
            pl.BlockSpec((BLK, D), lambda i, j: (j, 0)),
            pl.BlockSpec((D, HALF), lambda i, j: (0, i)),
            pl.BlockSpec((1, 1, HALF), lambda i, j: (i, 0, 0)),
        ],
        out_specs=pl.BlockSpec((BLK, HALF), lambda i, j: (i * NB + j, 0)),
        out_shape=jax.ShapeDtypeStruct((NC * N, HALF), jnp.float32),
    )(x, W1, b1_2h)


def _norm_mm_body(a0_ref, a1_ref, d0_ref, d1_ref, w_ref, b_ref, out_ref):
    deg = d0_ref[:, 0:1] + d1_ref[:, 0:1]
    inv = 1.0 / jnp.maximum(deg, 1.0)
    h0 = jnp.maximum(a0_ref[...] * inv, 0.0)
    h1 = jnp.maximum(a1_ref[...] * inv, 0.0)
    acc = jnp.dot(h0, w_ref[0:HALF, :], preferred_element_type=jnp.float32)
    acc += jnp.dot(h1, w_ref[HALF:D, :], preferred_element_type=jnp.float32)
    out_ref[...] = acc + b_ref[0]


def _support2(agg1, deg, W2, b2_2h):
    return pl.pallas_call(
        _norm_mm_body,
        grid=(NC, NB),
        in_specs=[
            pl.BlockSpec((BLK, HALF), lambda i, j: (j, 0)),
            pl.BlockSpec((BLK, HALF), lambda i, j: (NB + j, 0)),
            pl.BlockSpec((BLK, HALF), lambda i, j: (j, 0)),
            pl.BlockSpec((BLK, HALF), lambda i, j: (NB + j, 0)),
            pl.BlockSpec((D, HALF), lambda i, j: (0, i)),
            pl.BlockSpec((1, 1, HALF), lambda i, j: (i, 0, 0)),
        ],
        out_specs=pl.BlockSpec((BLK, HALF), lambda i, j: (i * NB + j, 0)),
        out_shape=jax.ShapeDtypeStruct((NC * N, HALF), jnp.float32),
    )(agg1, agg1, deg, deg, W2, b2_2h)


def _final_body(a0_ref, a1_ref, d0_ref, d1_ref, out_ref):
    deg = d0_ref[:, 0:1] + d1_ref[:, 0:1]
    inv = 1.0 / jnp.maximum(deg, 1.0)
    out_ref[:, 0:HALF] = a0_ref[...] * inv
    out_ref[:, HALF:D] = a1_ref[...] * inv


def _finalize(agg2, deg):
    return pl.pallas_call(
        _final_body,
        grid=(NB,),
        in_specs=[
            pl.BlockSpec((BLK, HALF), lambda j: (j, 0)),
            pl.BlockSpec((BLK, HALF), lambda j: (NB + j, 0)),
            pl.BlockSpec((BLK, HALF), lambda j: (j, 0)),
            pl.BlockSpec((BLK, HALF), lambda j: (NB + j, 0)),
        ],
        out_specs=pl.BlockSpec((BLK, D), lambda j: (j, 0)),
        out_shape=jax.ShapeDtypeStruct((N, D), jnp.float32),
    )(agg2, agg2, deg, deg)


def _zero_acc(zeros_hbm, acc_sh, sid):
    rbase = sid * SPAN
    pltpu.sync_copy(zeros_hbm.at[pl.ds(rbase, SPAN)],
                    acc_sh.at[pl.ds(rbase, SPAN)])

    @pl.when(sid == 0)
    def _():
        pltpu.sync_copy(zeros_hbm.at[pl.ds(SPAN * NS, NROWS - SPAN * NS)],
                        acc_sh.at[pl.ds(SPAN * NS, NROWS - SPAN * NS)])


def _write_acc(acc_sh, out_hbm, cid, sid):
    rbase = sid * SPAN
    pltpu.sync_copy(acc_sh.at[pl.ds(rbase, SPAN)],
                    out_hbm.at[pl.ds(cid * N + rbase, SPAN)])

    @pl.when(sid == 0)
    def _():
        pltpu.sync_copy(acc_sh.at[pl.ds(SPAN * NS, TAIL)],
                        out_hbm.at[pl.ds(cid * N + SPAN * NS, TAIL)])


def _sc_body_common(with_deg, sup_hbm, src3_hbm, dst3_hbm, dstd3_hbm,
                    zeros_hbm, ones_hbm, agg_out, deg_out,
                    src_v, buf0, buf1, dstc0_i, dstc1_i,
                    semg0, semg1, semd0, semd1, acc_sh):
    cid = lax.axis_index("c")
    sid = lax.axis_index("s")

    wid = cid * NS + sid

    if with_deg:
        pltpu.sync_copy(dstd3_hbm.at[pl.ds(wid * DNCH, DNCH)],
                        src_v.at[pl.ds(0, DNCH)])
        pltpu.sync_copy(ones_hbm, buf0)
    _zero_acc(zeros_hbm, acc_sh, sid)
    plsc.subcore_barrier()

    if with_deg:
        def deg_step(k, carry):
            pltpu.sync_copy(buf0, acc_sh.at[src_v.at[k]], add=True)
            return carry

        lax.fori_loop(0, DNCH, deg_step, 0)
        plsc.subcore_barrier()
        _write_acc(acc_sh, deg_out, cid, sid)
        _zero_acc(zeros_hbm, acc_sh, sid)
        plsc.subcore_barrier()

    pltpu.sync_copy(src3_hbm.at[pl.ds(wid * NCH, NCH)], src_v)

    def g_issue(k, buf, sem):
        pltpu.async_copy(sup_hbm.at[src_v.at[k]], buf, sem)

    def g_wait(buf, sem):
        pltpu.make_async_copy(sup_hbm.at[src_v.at[0]], buf, sem).wait()

    def d_issue(k, dbuf, sem):
        pltpu.async_copy(dst3_hbm.at[sid * NCH + k], dbuf, sem)

    def d_wait(dbuf, sem):
        pltpu.make_async_copy(dst3_hbm.at[sid * NCH], dbuf, sem).wait()

    d_issue(0, dstc0_i, semd0)
    g_issue(0, buf0, semg0)
    last = NCH - 1

    def pair_step(j, carry):
        k0 = 2 * j
        k1 = k0 + 1
        k2 = jnp.minimum(k0 + 2, last)
        d_issue(k1, dstc1_i, semd1)
        g_issue(k1, buf1, semg1)
        d_wait(dstc0_i, semd0)
        g_wait(buf0, semg0)
        pltpu.sync_copy(buf0, acc_sh.at[dstc0_i], add=True)
        d_issue(k2, dstc0_i, semd0)
        g_issue(k2, buf0, semg0)
        d_wait(dstc1_i, semd1)
        g_wait(buf1, semg1)
        pltpu.sync_copy(buf1, acc_sh.at[dstc1_i], add=True)
        return carry

    lax.fori_loop(0, NCH // 2, pair_step, 0)
    d_wait(dstc0_i, semd0)
    g_wait(buf0, semg0)
    plsc.subcore_barrier()
    _write_acc(acc_sh, agg_out, cid, sid)


def _make_sc_kernel(with_deg):
    mesh = plsc.VectorSubcoreMesh(core_axis_name="c", subcore_axis_name="s")
    if with_deg:
        out_type = (jax.ShapeDtypeStruct((NC * N, HALF), jnp.float32),
                    jax.ShapeDtypeStruct((NC * N, HALF), jnp.float32))
    else:
        out_type = jax.ShapeDtypeStruct((NC * N, HALF), jnp.float32)
    scratch = [
        pltpu.VMEM((NCH, CW), jnp.int32),
        pltpu.VMEM((CW, HALF), jnp.float32),
        pltpu.VMEM((CW, HALF), jnp.float32),
        pltpu.VMEM((CW,), jnp.int32),
        pltpu.VMEM((CW,), jnp.int32),
        pltpu.SemaphoreType.DMA,
        pltpu.SemaphoreType.DMA,
        pltpu.SemaphoreType.DMA,
        pltpu.SemaphoreType.DMA,
        pltpu.VMEM_SHARED((NROWS, HALF), jnp.float32),
    ]

    if with_deg:
        def body(sup, src3, dst3, dstd3, z, ones, agg_out, deg_out,
                 src_v, buf0, buf1, dstc0_i, dstc1_i,
                 semg0, semg1, semd0, semd1, acc_sh):
            _sc_body_common(True, sup, src3, dst3, dstd3, z, ones,
                            agg_out, deg_out,
                            src_v, buf0, buf1, dstc0_i, dstc1_i,
                            semg0, semg1, semd0, semd1, acc_sh)
    else:
        def body(sup, src3, dst3, dstd3, z, ones, agg_out,
                 src_v, buf0, buf1, dstc0_i, dstc1_i,
                 semg0, semg1, semd0, semd1, acc_sh):
            _sc_body_common(False, sup, src3, dst3, dstd3, z, ones,
                            agg_out, None,
                            src_v, buf0, buf1, dstc0_i, dstc1_i,
                            semg0, semg1, semd0, semd1, acc_sh)

    return pl.kernel(body, out_type=out_type, mesh=mesh, scratch_types=scratch)


_sc_layer1 = _make_sc_kernel(True)
_sc_layer2 = _make_sc_kernel(False)


def kernel(x, edge_index, W1, b1, W2, b2):
    src = edge_index[0]
    dst = edge_index[1]
    padw = NCH * CW - EDGES_PER_TILE
    sp = jnp.concatenate(
        [src.reshape(NS, EDGES_PER_TILE),
         jnp.zeros((NS, padw), jnp.int32)], axis=1)
    src3 = jnp.concatenate([sp, sp + N], axis=0).reshape(NC * NS * NCH, CW)
    dst3 = jnp.concatenate(
        [dst.reshape(NS, EDGES_PER_TILE),
         jnp.full((NS, padw), N, jnp.int32)], axis=1).reshape(NS * NCH, CW)
    dpadw = DNCH * CW - DEG_PER_WORKER
    dstd3 = jnp.concatenate(
        [dst.reshape(NC * NS, DEG_PER_WORKER),
         jnp.full((NC * NS, dpadw), N, jnp.int32)],
        axis=1).reshape(NC * NS * DNCH, CW)
    b1_2h = b1.reshape(NC, 1, HALF)
    b2_2h = b2.reshape(NC, 1, HALF)
    zeros = jnp.zeros((NROWS, HALF), jnp.float32)
    ones = jnp.ones((CW, HALF), jnp.float32)

    sup1 = _support1(x, W1, b1_2h)
    agg1, deg = _sc_layer1(sup1, src3, dst3, dstd3, zeros, ones)
    sup2 = _support2(agg1, deg, W2, b2_2h)
    agg2 = _sc_layer2(sup2, src3, dst3, dstd3, zeros, ones)
    return _finalize(agg2, deg)


# speedup vs baseline: 1.0276x; 1.0276x over previous
import jax
import jax.numpy as jnp
from jax import lax
from jax.experimental import pallas as pl
from jax.experimental.pallas import tpu as pltpu
from jax.experimental.pallas import tpu_sc as plsc

N = 10000
E = 160000
D = 256
HALF = 128
NC = 2
NS = 16
CW = 128
EDGES_PER_TILE = E // NS
NCH = 80
DEG_PER_WORKER = E // (NC * NS)
DNCH = 40
NROWS = N + 8
SPAN = 624
TAIL = N - SPAN * NS
BLK = 1000
NB = N // BLK


def _mm_body(x_ref, w_ref, b_ref, out_ref):
    acc = jnp.dot(x_ref[...], w_ref[...], preferred_element_type=jnp.float32)
    out_ref[...] = acc + b_ref[0]


def _support1(x, W1, b1_2h):
    return pl.pallas_call(
        _mm_body,
        grid=(NC, NB),
        in_specs=[
            pl.BlockSpec((BLK, D), lambda i, j: (j, 0)),
            pl.BlockSpec((D, HALF), lambda i, j: (0, i)),
            pl.BlockSpec((1, 1, HALF), lambda i, j: (i, 0, 0)),
        ],
        out_specs=pl.BlockSpec((BLK, HALF), lambda i, j: (i * NB + j, 0)),
        out_shape=jax.ShapeDtypeStruct((NC * N, HALF), jnp.float32),
    )(x, W1, b1_2h)


def _norm_mm_body(a0_ref, a1_ref, d0_ref, d1_ref, w_ref, b_ref, out_ref):
    deg = d0_ref[:, 0:1] + d1_ref[:, 0:1]
    inv = 1.0 / jnp.maximum(deg, 1.0)
    h0 = jnp.maximum(a0_ref[...] * inv, 0.0)
    h1 = jnp.maximum(a1_ref[...] * inv, 0.0)
    acc = jnp.dot(h0, w_ref[0:HALF, :], preferred_element_type=jnp.float32)
    acc += jnp.dot(h1, w_ref[HALF:D, :], preferred_element_type=jnp.float32)
    out_ref[...] = acc + b_ref[0]


def _support2(agg1, deg, W2, b2_2h):
    return pl.pallas_call(
        _norm_mm_body,
        grid=(NC, NB),
        in_specs=[
            pl.BlockSpec((BLK, HALF), lambda i, j: (j, 0)),
            pl.BlockSpec((BLK, HALF), lambda i, j: (NB + j, 0)),
            pl.BlockSpec((BLK, HALF), lambda i, j: (j, 0)),
            pl.BlockSpec((BLK, HALF), lambda i, j: (NB + j, 0)),
            pl.BlockSpec((D, HALF), lambda i, j: (0, i)),
            pl.BlockSpec((1, 1, HALF), lambda i, j: (i, 0, 0)),
        ],
        out_specs=pl.BlockSpec((BLK, HALF), lambda i, j: (i * NB + j, 0)),
        out_shape=jax.ShapeDtypeStruct((NC * N, HALF), jnp.float32),
    )(agg1, agg1, deg, deg, W2, b2_2h)


def _final_body(a0_ref, a1_ref, d0_ref, d1_ref, out_ref):
    deg = d0_ref[:, 0:1] + d1_ref[:, 0:1]
    inv = 1.0 / jnp.maximum(deg, 1.0)
    out_ref[:, 0:HALF] = a0_ref[...] * inv
    out_ref[:, HALF:D] = a1_ref[...] * inv


def _finalize(agg2, deg):
    return pl.pallas_call(
        _final_body,
        grid=(NB,),
        in_specs=[
            pl.BlockSpec((BLK, HALF), lambda j: (j, 0)),
            pl.BlockSpec((BLK, HALF), lambda j: (NB + j, 0)),
            pl.BlockSpec((BLK, HALF), lambda j: (j, 0)),
            pl.BlockSpec((BLK, HALF), lambda j: (NB + j, 0)),
        ],
        out_specs=pl.BlockSpec((BLK, D), lambda j: (j, 0)),
        out_shape=jax.ShapeDtypeStruct((N, D), jnp.float32),
    )(agg2, agg2, deg, deg)


def _zero_acc(zeros_hbm, acc_sh, sid):
    rbase = sid * SPAN
    pltpu.sync_copy(zeros_hbm.at[pl.ds(rbase, SPAN)],
                    acc_sh.at[pl.ds(rbase, SPAN)])

    @pl.when(sid == 0)
    def _():
        pltpu.sync_copy(zeros_hbm.at[pl.ds(SPAN * NS, NROWS - SPAN * NS)],
                        acc_sh.at[pl.ds(SPAN * NS, NROWS - SPAN * NS)])


def _write_acc(acc_sh, out_hbm, cid, sid):
    rbase = sid * SPAN
    pltpu.sync_copy(acc_sh.at[pl.ds(rbase, SPAN)],
                    out_hbm.at[pl.ds(cid * N + rbase, SPAN)])

    @pl.when(sid == 0)
    def _():
        pltpu.sync_copy(acc_sh.at[pl.ds(SPAN * NS, TAIL)],
                        out_hbm.at[pl.ds(cid * N + SPAN * NS, TAIL)])


def _sc_body_common(with_deg, sup_hbm, src3_hbm, dst3_hbm, dstd3_hbm,
                    zeros_hbm, ones_hbm, agg_out, deg_out,
                    src_v, dst_v, buf0, acc_sh):
    cid = lax.axis_index("c")
    sid = lax.axis_index("s")

    wid = cid * NS + sid

    if with_deg:
        pltpu.sync_copy(dstd3_hbm.at[pl.ds(wid * DNCH, DNCH)],
                        src_v.at[pl.ds(0, DNCH)])
        pltpu.sync_copy(ones_hbm, buf0)
    _zero_acc(zeros_hbm, acc_sh, sid)
    plsc.subcore_barrier()

    if with_deg:
        def deg_step(k, carry):
            pltpu.sync_copy(buf0, acc_sh.at[src_v.at[k]], add=True)
            return carry

        lax.fori_loop(0, DNCH, deg_step, 0)
        plsc.subcore_barrier()
        _write_acc(acc_sh, deg_out, cid, sid)
        _zero_acc(zeros_hbm, acc_sh, sid)
        plsc.subcore_barrier()

    pltpu.sync_copy(src3_hbm.at[pl.ds(wid * NCH, NCH)], src_v)

    pltpu.sync_copy(dst3_hbm.at[pl.ds(sid * NCH, NCH)], dst_v)

    def chunk_step(k, carry):
        pltpu.sync_copy(sup_hbm.at[src_v.at[k]], buf0)
        pltpu.sync_copy(buf0, acc_sh.at[dst_v.at[k]], add=True)
        return carry

    lax.fori_loop(0, NCH, chunk_step, 0)
    plsc.subcore_barrier()
    _write_acc(acc_sh, agg_out, cid, sid)


def _make_sc_kernel(with_deg):
    mesh = plsc.VectorSubcoreMesh(core_axis_name="c", subcore_axis_name="s")
    if with_deg:
        out_type = (jax.ShapeDtypeStruct((NC * N, HALF), jnp.float32),
                    jax.ShapeDtypeStruct((NC * N, HALF), jnp.float32))
    else:
        out_type = jax.ShapeDtypeStruct((NC * N, HALF), jnp.float32)
    scratch = [
        pltpu.VMEM((NCH, CW), jnp.int32),
        pltpu.VMEM((NCH, CW), jnp.int32),
        pltpu.VMEM((CW, HALF), jnp.float32),
        pltpu.VMEM_SHARED((NROWS, HALF), jnp.float32),
    ]

    if with_deg:
        def body(sup, src3, dst3, dstd3, z, ones, agg_out, deg_out,
                 src_v, dst_v, buf0, acc_sh):
            _sc_body_common(True, sup, src3, dst3, dstd3, z, ones,
                            agg_out, deg_out, src_v, dst_v, buf0, acc_sh)
    else:
        def body(sup, src3, dst3, dstd3, z, ones, agg_out,
                 src_v, dst_v, buf0, acc_sh):
            _sc_body_common(False, sup, src3, dst3, dstd3, z, ones,
                            agg_out, None, src_v, dst_v, buf0, acc_sh)

    return pl.kernel(body, out_type=out_type, mesh=mesh, scratch_types=scratch)


_sc_layer1 = _make_sc_kernel(True)
_sc_layer2 = _make_sc_kernel(False)


def kernel(x, edge_index, W1, b1, W2, b2):
    src = edge_index[0]
    dst = edge_index[1]
    padw = NCH * CW - EDGES_PER_TILE
    sp = jnp.concatenate(
        [src.reshape(NS, EDGES_PER_TILE),
         jnp.zeros((NS, padw), jnp.int32)], axis=1)
    src3 = jnp.concatenate([sp, sp + N], axis=0).reshape(NC * NS * NCH, CW)
    dst3 = jnp.concatenate(
        [dst.reshape(NS, EDGES_PER_TILE),
         jnp.full((NS, padw), N, jnp.int32)], axis=1).reshape(NS * NCH, CW)
    dpadw = DNCH * CW - DEG_PER_WORKER
    dstd3 = jnp.concatenate(
        [dst.reshape(NC * NS, DEG_PER_WORKER),
         jnp.full((NC * NS, dpadw), N, jnp.int32)],
        axis=1).reshape(NC * NS * DNCH, CW)
    b1_2h = b1.reshape(NC, 1, HALF)
    b2_2h = b2.reshape(NC, 1, HALF)
    zeros = jnp.zeros((NROWS, HALF), jnp.float32)
    ones = jnp.ones((CW, HALF), jnp.float32)

    sup1 = _support1(x, W1, b1_2h)
    agg1, deg = _sc_layer1(sup1, src3, dst3, dstd3, zeros, ones)
    sup2 = _support2(agg1, deg, W2, b2_2h)
    agg2 = _sc_layer2(sup2, src3, dst3, dstd3, zeros, ones)
    return _finalize(agg2, deg)


# speedup vs baseline: 1.0305x; 1.0028x over previous
import jax
import jax.numpy as jnp
from jax import lax
from jax.experimental import pallas as pl
from jax.experimental.pallas import tpu as pltpu
from jax.experimental.pallas import tpu_sc as plsc

N = 10000
E = 160000
D = 256
HALF = 128
NC = 2
NS = 16
CW = 128
EDGES_PER_TILE = E // NS
NCH = 80
DEG_PER_WORKER = E // (NC * NS)
DNCH = 40
NROWS = N + 8
SPAN = 624
TAIL = N - SPAN * NS
BLK = 1000
NB = N // BLK


def _mm_body(x_ref, w_ref, b_ref, out_ref):
    acc = jnp.dot(x_ref[...], w_ref[...], preferred_element_type=jnp.float32)
    out_ref[...] = acc + b_ref[0]


def _support1(x, W1, b1_2h):
    return pl.pallas_call(
        _mm_body,
        grid=(NC, NB),
        in_specs=[
            pl.BlockSpec((BLK, D), lambda i, j: (j, 0)),
            pl.BlockSpec((D, HALF), lambda i, j: (0, i)),
            pl.BlockSpec((1, 1, HALF), lambda i, j: (i, 0, 0)),
        ],
        out_specs=pl.BlockSpec((BLK, HALF), lambda i, j: (i * NB + j, 0)),
        out_shape=jax.ShapeDtypeStruct((NC * N, HALF), jnp.float32),
    )(x, W1, b1_2h)


def _norm_mm_body(a0_ref, a1_ref, d0_ref, d1_ref, w_ref, b_ref, out_ref):
    d0 = d0_ref[:, 0:1]
    d1 = d1_ref[:, 0:1]
    deg = d0 + d1
    inv = 1.0 / jnp.maximum(deg, 1.0)
    h0 = jnp.maximum((a0_ref[...] - d0) * inv, 0.0)
    h1 = jnp.maximum((a1_ref[...] - d1) * inv, 0.0)
    acc = jnp.dot(h0, w_ref[0:HALF, :], preferred_element_type=jnp.float32)
    acc += jnp.dot(h1, w_ref[HALF:D, :], preferred_element_type=jnp.float32)
    out_ref[...] = acc + b_ref[0]


def _support2(agg1, deg, W2, b2_2h):
    return pl.pallas_call(
        _norm_mm_body,
        grid=(NC, NB),
        in_specs=[
            pl.BlockSpec((BLK, HALF), lambda i, j: (j, 0)),
            pl.BlockSpec((BLK, HALF), lambda i, j: (NB + j, 0)),
            pl.BlockSpec((BLK, HALF), lambda i, j: (j, 0)),
            pl.BlockSpec((BLK, HALF), lambda i, j: (NB + j, 0)),
            pl.BlockSpec((D, HALF), lambda i, j: (0, i)),
            pl.BlockSpec((1, 1, HALF), lambda i, j: (i, 0, 0)),
        ],
        out_specs=pl.BlockSpec((BLK, HALF), lambda i, j: (i * NB + j, 0)),
        out_shape=jax.ShapeDtypeStruct((NC * N, HALF), jnp.float32),
    )(agg1, agg1, deg, deg, W2, b2_2h)


def _final_body(a0_ref, a1_ref, d0_ref, d1_ref, out_ref):
    deg = d0_ref[:, 0:1] + d1_ref[:, 0:1]
    inv = 1.0 / jnp.maximum(deg, 1.0)
    out_ref[:, 0:HALF] = a0_ref[...] * inv
    out_ref[:, HALF:D] = a1_ref[...] * inv


def _finalize(agg2, deg):
    return pl.pallas_call(
        _final_body,
        grid=(NB,),
        in_specs=[
            pl.BlockSpec((BLK, HALF), lambda j: (j, 0)),
            pl.BlockSpec((BLK, HALF), lambda j: (NB + j, 0)),
            pl.BlockSpec((BLK, HALF), lambda j: (j, 0)),
            pl.BlockSpec((BLK, HALF), lambda j: (NB + j, 0)),
        ],
        out_specs=pl.BlockSpec((BLK, D), lambda j: (j, 0)),
        out_shape=jax.ShapeDtypeStruct((N, D), jnp.float32),
    )(agg2, agg2, deg, deg)


def _zero_acc(zeros_hbm, acc_sh, sid):
    rbase = sid * SPAN
    pltpu.sync_copy(zeros_hbm.at[pl.ds(rbase, SPAN)],
                    acc_sh.at[pl.ds(rbase, SPAN)])

    @pl.when(sid == 0)
    def _():
        pltpu.sync_copy(zeros_hbm.at[pl.ds(SPAN * NS, NROWS - SPAN * NS)],
                        acc_sh.at[pl.ds(SPAN * NS, NROWS - SPAN * NS)])


def _write_acc(acc_sh, out_hbm, cid, sid):
    rbase = sid * SPAN
    pltpu.sync_copy(acc_sh.at[pl.ds(rbase, SPAN)],
                    out_hbm.at[pl.ds(cid * N + rbase, SPAN)])

    @pl.when(sid == 0)
    def _():
        pltpu.sync_copy(acc_sh.at[pl.ds(SPAN * NS, TAIL)],
                        out_hbm.at[pl.ds(cid * N + SPAN * NS, TAIL)])


def _sc_body_common(with_deg, sup_hbm, src3_hbm, dst3_hbm, dstd3_hbm,
                    zeros_hbm, ones_hbm, agg_out, deg_out,
                    src_v, dst_v, buf0, acc_sh):
    cid = lax.axis_index("c")
    sid = lax.axis_index("s")

    wid = cid * NS + sid

    if with_deg:
        pltpu.sync_copy(dstd3_hbm.at[pl.ds(wid * DNCH, DNCH)],
                        src_v.at[pl.ds(0, DNCH)])
        pltpu.sync_copy(ones_hbm, buf0)
    _zero_acc(zeros_hbm, acc_sh, sid)
    plsc.subcore_barrier()

    if with_deg:
        def deg_step(k, carry):
            pltpu.sync_copy(buf0, acc_sh.at[src_v.at[k]], add=True)
            return carry

        lax.fori_loop(0, DNCH, deg_step, 0)
        plsc.subcore_barrier()
        _write_acc(acc_sh, deg_out, cid, sid)
        plsc.subcore_barrier()

    pltpu.sync_copy(src3_hbm.at[pl.ds(wid * NCH, NCH)], src_v)

    pltpu.sync_copy(dst3_hbm.at[pl.ds(sid * NCH, NCH)], dst_v)

    def chunk_step(k, carry):
        pltpu.sync_copy(sup_hbm.at[src_v.at[k]], buf0)
        pltpu.sync_copy(buf0, acc_sh.at[dst_v.at[k]], add=True)
        return carry

    lax.fori_loop(0, NCH, chunk_step, 0)
    plsc.subcore_barrier()
    _write_acc(acc_sh, agg_out, cid, sid)


def _make_sc_kernel(with_deg):
    mesh = plsc.VectorSubcoreMesh(core_axis_name="c", subcore_axis_name="s")
    if with_deg:
        out_type = (jax.ShapeDtypeStruct((NC * N, HALF), jnp.float32),
                    jax.ShapeDtypeStruct((NC * N, HALF), jnp.float32))
    else:
        out_type = jax.ShapeDtypeStruct((NC * N, HALF), jnp.float32)
    scratch = [
        pltpu.VMEM((NCH, CW), jnp.int32),
        pltpu.VMEM((NCH, CW), jnp.int32),
        pltpu.VMEM((CW, HALF), jnp.float32),
        pltpu.VMEM_SHARED((NROWS, HALF), jnp.float32),
    ]

    if with_deg:
        def body(sup, src3, dst3, dstd3, z, ones, agg_out, deg_out,
                 src_v, dst_v, buf0, acc_sh):
            _sc_body_common(True, sup, src3, dst3, dstd3, z, ones,
                            agg_out, deg_out, src_v, dst_v, buf0, acc_sh)
    else:
        def body(sup, src3, dst3, dstd3, z, ones, agg_out,
                 src_v, dst_v, buf0, acc_sh):
            _sc_body_common(False, sup, src3, dst3, dstd3, z, ones,
                            agg_out, None, src_v, dst_v, buf0, acc_sh)

    return pl.kernel(body, out_type=out_type, mesh=mesh, scratch_types=scratch)


_sc_layer1 = _make_sc_kernel(True)
_sc_layer2 = _make_sc_kernel(False)


def kernel(x, edge_index, W1, b1, W2, b2):
    src = edge_index[0]
    dst = edge_index[1]
    padw = NCH * CW - EDGES_PER_TILE
    sp = jnp.concatenate(
        [src.reshape(NS, EDGES_PER_TILE),
         jnp.zeros((NS, padw), jnp.int32)], axis=1)
    src3 = jnp.concatenate([sp, sp + N], axis=0).reshape(NC * NS * NCH, CW)
    dst3 = jnp.concatenate(
        [dst.reshape(NS, EDGES_PER_TILE),
         jnp.full((NS, padw), N, jnp.int32)], axis=1).reshape(NS * NCH, CW)
    dpadw = DNCH * CW - DEG_PER_WORKER
    dstd3 = jnp.concatenate(
        [dst.reshape(NC * NS, DEG_PER_WORKER),
         jnp.full((NC * NS, dpadw), N, jnp.int32)],
        axis=1).reshape(NC * NS * DNCH, CW)
    b1_2h = b1.reshape(NC, 1, HALF)
    b2_2h = b2.reshape(NC, 1, HALF)
    zeros = jnp.zeros((NROWS, HALF), jnp.float32)
    ones = jnp.ones((CW, HALF), jnp.float32)

    sup1 = _support1(x, W1, b1_2h)
    agg1, deg = _sc_layer1(sup1, src3, dst3, dstd3, zeros, ones)
    sup2 = _support2(agg1, deg, W2, b2_2h)
    agg2 = _sc_layer2(sup2, src3, dst3, dstd3, zeros, ones)
    return _finalize(agg2, deg)
